# baseline (speedup 1.0000x reference)
import jax
import jax.numpy as jnp
from jax import lax
from jax.experimental import pallas as pl
from jax.experimental.pallas import tpu as pltpu
from jax.experimental.pallas import tpu_sc as plsc

SEQ = 200
DIM = 32
BATCH = 4096
VOCAB = 1000000
NW = 32
B_BLK = BATCH // NW
NEB = DIM // 8
SCALE = float(DIM) ** 0.5


def _body(idx_hbm, table_hbm, pos_hbm, out_hbm, idxr_v, idxt_v, pos_v, rows_v, out_t, sem):
    cidx = lax.axis_index("c")
    sidx = lax.axis_index("s")
    w = sidx * 2 + cidx
    b0 = pl.multiple_of(w * B_BLK, 8)

    pltpu.sync_copy(idx_hbm.at[pl.ds(b0, B_BLK)], idxr_v)
    pltpu.sync_copy(pos_hbm, pos_v)

    iota = jnp.arange(16, dtype=jnp.int32)

    def tr_body(s, carry):
        svec = jnp.full((16,), 0, jnp.int32) + s
        for g in range(B_BLK // 16):
            v = plsc.load_gather(idxr_v, [iota + 16 * g, svec])
            idxt_v[s, pl.ds(16 * g, 16)] = v
        return carry

    lax.fori_loop(0, SEQ, tr_body, 0)

    def item(s, carry):
        pltpu.async_copy(table_hbm.at[idxt_v.at[s]], rows_v, sem).wait()
        svec = jnp.full((16,), 0, jnp.int32) + s
        for e in range(DIM):
            evec = jnp.full((16,), e, jnp.int32)
            p = plsc.load_gather(pos_v, [svec, evec])
            for g in range(B_BLK // 16):
                v = plsc.load_gather(rows_v, [iota + 16 * g, evec])
                out_t[e // 8, e % 8, pl.ds(16 * g, 16)] = v * SCALE + p
        for eb in range(NEB):
            pltpu.sync_copy(out_t.at[eb], out_hbm.at[s, eb, w])
        return carry

    lax.fori_loop(0, SEQ, item, 0)


@jax.jit
def kernel(inputs, token_table, pos_table):
    tt = lax.optimization_barrier(token_table.reshape(VOCAB * DIM // 128, 128))
    tt = tt.reshape(VOCAB, DIM)
    mesh = plsc.VectorSubcoreMesh(core_axis_name="c", subcore_axis_name="s")
    out5 = pl.kernel(
        _body,
        out_type=jax.ShapeDtypeStruct((SEQ, NEB, BATCH // 128, 8, 128), jnp.float32),
        mesh=mesh,
        scratch_types=[
            pltpu.VMEM((B_BLK, SEQ), jnp.int32),
            pltpu.VMEM((SEQ, B_BLK), jnp.int32),
            pltpu.VMEM((SEQ, DIM), jnp.float32),
            pltpu.VMEM((B_BLK, DIM), jnp.float32),
            pltpu.VMEM((NEB, 8, 128), jnp.float32),
            pltpu.SemaphoreType.DMA,
        ],
        compiler_params=pltpu.CompilerParams(
            use_tc_tiling_on_sc=False, needs_layout_passes=False
        ),
    )(inputs, tt, pos_table)
    return out5.transpose(2, 4, 0, 1, 3).reshape(BATCH, SEQ, DIM)


# speedup vs baseline: 1.0743x; 1.0743x over previous
import jax
import jax.numpy as jnp
from jax import lax
from jax.experimental import pallas as pl
from jax.experimental.pallas import tpu as pltpu
from jax.experimental.pallas import tpu_sc as plsc

SEQ = 200
DIM = 32
BATCH = 4096
VOCAB = 1000000
NW = 32
B_BLK = BATCH // NW
NEB = DIM // 8
SCALE = float(DIM) ** 0.5


NBUF = 4
NGRP = SEQ // NBUF


def _body(idx_hbm, table_hbm, pos_hbm, out_hbm, idxr_v, idxt_v, pos_v, rows_v, out_t, gsem, osem):
    cidx = lax.axis_index("c")
    sidx = lax.axis_index("s")
    w = sidx * 2 + cidx
    b0 = pl.multiple_of(w * B_BLK, 8)

    pltpu.sync_copy(idx_hbm.at[pl.ds(b0, B_BLK)], idxr_v)
    pltpu.sync_copy(pos_hbm, pos_v)

    iota = jnp.arange(16, dtype=jnp.int32)

    def tr_body(s, carry):
        svec = jnp.full((16,), 0, jnp.int32) + s
        for g in range(B_BLK // 16):
            v = plsc.load_gather(idxr_v, [iota + 16 * g, svec])
            idxt_v[s, pl.ds(16 * g, 16)] = v
        return carry

    lax.fori_loop(0, SEQ, tr_body, 0)

    def gather_start(s, slot):
        pltpu.async_copy(table_hbm.at[idxt_v.at[s]], rows_v.at[slot], gsem.at[slot])

    def gather_drain(slot):
        pltpu.make_async_copy(
            table_hbm.at[pl.ds(0, B_BLK)], rows_v.at[slot], gsem.at[slot]
        ).wait()

    def out_group_start(g, par):
        pltpu.async_copy(out_t.at[par], out_hbm.at[pl.ds(NBUF * g, NBUF), :, w], osem.at[par])

    def out_group_drain(par):
        pltpu.make_async_copy(
            out_t.at[par], out_hbm.at[pl.ds(0, NBUF), :, 0], osem.at[par]
        ).wait()

    for b in range(NBUF):
        gather_start(b, b)

    def group(g, carry):
        par = lax.rem(g, 2)
        @pl.when(g >= 2)
        def _():
            out_group_drain(par)

        for b in range(NBUF):
            s = NBUF * g + b
            gather_drain(b)
            svec = jnp.full((16,), 0, jnp.int32) + s
            for e in range(DIM):
                evec = jnp.full((16,), e, jnp.int32)
                p = plsc.load_gather(pos_v, [svec, evec])
                for q in range(B_BLK // 16):
                    v = plsc.load_gather(rows_v.at[b], [iota + 16 * q, evec])
                    out_t[par, b, e // 8, e % 8, pl.ds(16 * q, 16)] = v * SCALE + p
            @pl.when(g < NGRP - 1)
            def _():
                gather_start(s + NBUF, b)

        out_group_start(g, par)
        return carry

    lax.fori_loop(0, NGRP, group, 0)
    out_group_drain(0)
    out_group_drain(1)


@jax.jit
def kernel(inputs, token_table, pos_table):
    tt = lax.optimization_barrier(token_table.reshape(VOCAB * DIM // 128, 128))
    tt = tt.reshape(VOCAB, DIM)
    mesh = plsc.VectorSubcoreMesh(core_axis_name="c", subcore_axis_name="s")
    out5 = pl.kernel(
        _body,
        out_type=jax.ShapeDtypeStruct((SEQ, NEB, BATCH // 128, 8, 128), jnp.float32),
        mesh=mesh,
        scratch_types=[
            pltpu.VMEM((B_BLK, SEQ), jnp.int32),
            pltpu.VMEM((SEQ, B_BLK), jnp.int32),
            pltpu.VMEM((SEQ, DIM), jnp.float32),
            pltpu.VMEM((NBUF, B_BLK, DIM), jnp.float32),
            pltpu.VMEM((2, NBUF, NEB, 8, 128), jnp.float32),
            pltpu.SemaphoreType.DMA((NBUF,)),
            pltpu.SemaphoreType.DMA((2,)),
        ],
        compiler_params=pltpu.CompilerParams(
            use_tc_tiling_on_sc=False, needs_layout_passes=False
        ),
    )(inputs, tt, pos_table)
    return out5.transpose(2, 4, 0, 1, 3).reshape(BATCH, SEQ, DIM)


# speedup vs baseline: 1.5385x; 1.4321x over previous
import jax
import jax.numpy as jnp
from jax import lax
from jax.experimental import pallas as pl
from jax.experimental.pallas import tpu as pltpu
from jax.experimental.pallas import tpu_sc as plsc

SEQ = 200
DIM = 32
BATCH = 4096
VOCAB = 1000000
NW = 32
B_BLK = BATCH // NW
NEB = DIM // 8
SCALE = float(DIM) ** 0.5


NBUF = 4
NGRP = SEQ // NBUF


def _body(idx_hbm, table_hbm, pos_hbm, out_hbm, idxr_v, idxt_v, pos_v, rows_v, out_t, gsem, osem):
    cidx = lax.axis_index("c")
    sidx = lax.axis_index("s")
    w = sidx * 2 + cidx
    b0 = pl.multiple_of(w * B_BLK, 8)

    pltpu.sync_copy(idx_hbm.at[pl.ds(b0, B_BLK)], idxr_v)
    pltpu.sync_copy(pos_hbm, pos_v)

    iota = jnp.arange(16, dtype=jnp.int32)
    eb_lo = iota // 8
    e8_lo = iota % 8
    eb_hi = (iota + 16) // 8
    e8_hi = (iota + 16) % 8

    def tr_body(s, carry):
        svec = jnp.full((16,), 0, jnp.int32) + s
        for g in range(B_BLK // 16):
            v = plsc.load_gather(idxr_v, [iota + 16 * g, svec])
            idxt_v[s, pl.ds(16 * g, 16)] = v
        return carry

    lax.fori_loop(0, SEQ, tr_body, 0)

    def gather_start(s, slot):
        pltpu.async_copy(table_hbm.at[idxt_v.at[s]], rows_v.at[slot], gsem.at[slot])

    def gather_drain(slot):
        pltpu.make_async_copy(
            table_hbm.at[pl.ds(0, B_BLK)], rows_v.at[slot], gsem.at[slot]
        ).wait()

    def out_group_start(g, par):
        pltpu.async_copy(
            out_t.at[par, :, :, :, pl.ds(0, 128)],
            out_hbm.at[pl.ds(NBUF * g, NBUF), :, w],
            osem.at[par],
        )

    def out_group_drain(par):
        pltpu.make_async_copy(
            out_t.at[par, :, :, :, pl.ds(0, 128)],
            out_hbm.at[pl.ds(0, NBUF), :, 0],
            osem.at[par],
        ).wait()

    for b in range(NBUF):
        gather_start(b, b)

    def group(g, carry):
        par = lax.rem(g, 2)
        @pl.when(g >= 2)
        def _():
            out_group_drain(par)

        for b in range(NBUF):
            s = NBUF * g + b
            gather_drain(b)
            p0 = pos_v[s, pl.ds(0, 16)]
            p1 = pos_v[s, pl.ds(16, 16)]
            tile = out_t.at[par, b]
            for bl in range(B_BLK):
                blv = jnp.full((16,), bl, jnp.int32)
                v0 = rows_v[b, bl, pl.ds(0, 16)]
                v1 = rows_v[b, bl, pl.ds(16, 16)]
                plsc.store_scatter(tile, [eb_lo, e8_lo, blv], v0 * SCALE + p0)
                plsc.store_scatter(tile, [eb_hi, e8_hi, blv], v1 * SCALE + p1)
            @pl.when(g < NGRP - 1)
            def _():
                gather_start(s + NBUF, b)

        out_group_start(g, par)
        return carry

    lax.fori_loop(0, NGRP, group, 0)
    out_group_drain(0)
    out_group_drain(1)


@jax.jit
def kernel(inputs, token_table, pos_table):
    tt = lax.optimization_barrier(token_table.reshape(VOCAB * DIM // 128, 128))
    tt = tt.reshape(VOCAB, DIM)
    mesh = plsc.VectorSubcoreMesh(core_axis_name="c", subcore_axis_name="s")
    out5 = pl.kernel(
        _body,
        out_type=jax.ShapeDtypeStruct((SEQ, NEB, BATCH // 128, 8, 128), jnp.float32),
        mesh=mesh,
        scratch_types=[
            pltpu.VMEM((B_BLK, SEQ), jnp.int32),
            pltpu.VMEM((SEQ, B_BLK), jnp.int32),
            pltpu.VMEM((SEQ, DIM), jnp.float32),
            pltpu.VMEM((NBUF, B_BLK, DIM), jnp.float32),
            pltpu.VMEM((2, NBUF, NEB, 8, 129), jnp.float32),
            pltpu.SemaphoreType.DMA((NBUF,)),
            pltpu.SemaphoreType.DMA((2,)),
        ],
        compiler_params=pltpu.CompilerParams(
            use_tc_tiling_on_sc=False, needs_layout_passes=False
        ),
    )(inputs, tt, pos_table)
    return out5.transpose(2, 4, 0, 1, 3).reshape(BATCH, SEQ, DIM)


# speedup vs baseline: 2.4810x; 1.6127x over previous
import jax
import jax.numpy as jnp
from jax import lax
from jax.experimental import pallas as pl
from jax.experimental.pallas import tpu as pltpu
from jax.experimental.pallas import tpu_sc as plsc

SEQ = 200
DIM = 32
BATCH = 4096
VOCAB = 1000000
NW = 32
B_BLK = BATCH // NW
NEB = DIM // 8
SCALE = float(DIM) ** 0.5


NBUF = 4
NGRP = SEQ // NBUF


def _body(idx_hbm, table_hbm, pos_hbm, out_hbm, idxr_v, idxt_v, pos_v, rows_v, out_t, gsem, osem):
    cidx = lax.axis_index("c")
    sidx = lax.axis_index("s")
    w = sidx * 2 + cidx
    b0 = pl.multiple_of(w * B_BLK, 8)

    pltpu.sync_copy(idx_hbm.at[pl.ds(b0, B_BLK)], idxr_v)
    pltpu.sync_copy(pos_hbm, pos_v)

    iota = jnp.arange(16, dtype=jnp.int32)
    eb_lo = iota // 8
    e8_lo = iota % 8
    eb_hi = (iota + 16) // 8
    e8_hi = (iota + 16) % 8

    def tr_body(s, carry):
        svec = jnp.full((16,), 0, jnp.int32) + s
        for g in range(B_BLK // 16):
            v = plsc.load_gather(idxr_v, [iota + 16 * g, svec])
            idxt_v[s, pl.ds(16 * g, 16)] = v
        return carry

    lax.fori_loop(0, SEQ, tr_body, 0)

    def gather_start(s, slot):
        pltpu.async_copy(table_hbm.at[idxt_v.at[s]], rows_v.at[slot], gsem.at[slot])

    def gather_drain(slot):
        pltpu.make_async_copy(
            table_hbm.at[pl.ds(0, B_BLK)], rows_v.at[slot], gsem.at[slot]
        ).wait()

    def out_group_start(g, par):
        pltpu.async_copy(
            out_t.at[par, :, :, :, pl.ds(0, 128)],
            out_hbm.at[pl.ds(NBUF * g, NBUF), :, w],
            osem.at[par],
        )

    def out_group_drain(par):
        pltpu.make_async_copy(
            out_t.at[par, :, :, :, pl.ds(0, 128)],
            out_hbm.at[pl.ds(0, NBUF), :, 0],
            osem.at[par],
        ).wait()

    for b in range(NBUF):
        gather_start(b, b)

    def group(g, carry):
        par = lax.rem(g, 2)
        @pl.when(g >= 2)
        def _():
            out_group_drain(par)

        for b in range(NBUF):
            s = NBUF * g + b
            gather_drain(b)
            p0 = pos_v[s, pl.ds(0, 16)]
            p1 = pos_v[s, pl.ds(16, 16)]
            tile = out_t.at[par, b]
            @plsc.parallel_loop(0, B_BLK, step=1, unroll=8)
            def _(bl):
                blv = jnp.full((16,), 0, jnp.int32) + bl
                v0 = rows_v[b, bl, pl.ds(0, 16)]
                v1 = rows_v[b, bl, pl.ds(16, 16)]
                plsc.store_scatter(tile, [eb_lo, e8_lo, blv], v0 * SCALE + p0)
                plsc.store_scatter(tile, [eb_hi, e8_hi, blv], v1 * SCALE + p1)
            @pl.when(g < NGRP - 1)
            def _():
                gather_start(s + NBUF, b)

        out_group_start(g, par)
        return carry

    lax.fori_loop(0, NGRP, group, 0)
    out_group_drain(0)
    out_group_drain(1)


@jax.jit
def kernel(inputs, token_table, pos_table):
    tt = token_table
    mesh = plsc.VectorSubcoreMesh(core_axis_name="c", subcore_axis_name="s")
    out5 = pl.kernel(
        _body,
        out_type=jax.ShapeDtypeStruct((SEQ, NEB, BATCH // 128, 8, 128), jnp.float32),
        mesh=mesh,
        scratch_types=[
            pltpu.VMEM((B_BLK, SEQ), jnp.int32),
            pltpu.VMEM((SEQ, B_BLK), jnp.int32),
            pltpu.VMEM((SEQ, DIM), jnp.float32),
            pltpu.VMEM((NBUF, B_BLK, DIM), jnp.float32),
            pltpu.VMEM((2, NBUF, NEB, 8, 129), jnp.float32),
            pltpu.SemaphoreType.DMA((NBUF,)),
            pltpu.SemaphoreType.DMA((2,)),
        ],
        compiler_params=pltpu.CompilerParams(
            use_tc_tiling_on_sc=False, needs_layout_passes=False
        ),
    )(inputs, tt, pos_table)
    return out5.transpose(2, 4, 0, 1, 3).reshape(BATCH, SEQ, DIM)
